# baseline (speedup 1.0000x reference)
import jax
import jax.numpy as jnp
from jax.experimental import pallas as pl

_B = 2048
_E = 64


def _body(xt_ref, c_ref, r_ref, out_ref, idx_ref):
    xt = xt_ref[...]
    c = c_ref[...]
    c_norm = jnp.sum(c * c, axis=1, keepdims=True)
    g = jax.lax.dot_general(
        c, xt, (((1,), (0,)), ((), ())),
        preferred_element_type=jnp.float32,
        precision=jax.lax.Precision.HIGHEST)
    scores = c_norm - 2.0 * g
    m = jnp.min(scores, axis=0, keepdims=True)
    subl = jax.lax.broadcasted_iota(jnp.int32, scores.shape, 0)
    idxrow = jnp.min(jnp.where(scores == m, subl, _E), axis=0,
                     keepdims=True)
    onehot_t = (subl == idxrow).astype(jnp.float32)
    w = 0.1 * jax.nn.sigmoid(r_ref[...])
    rhs = jnp.concatenate([w * c, w], axis=1)
    p = jax.lax.dot_general(
        rhs, onehot_t, (((0,), (0,)), ((), ())),
        preferred_element_type=jnp.float32,
        precision=jax.lax.Precision.DEFAULT)
    out_ref[...] = xt + (p[:_E, :] - p[_E:, :] * xt)
    idx_ref[...] = idxrow[:, None, :]


@jax.jit
def kernel(expert_activations, attractor_centers, attraction_radii):
    batch, e = expert_activations.shape
    xt = expert_activations.T
    grid = batch // _B
    r2d = attraction_radii[:, None]
    out_t, idx2 = pl.pallas_call(
        _body,
        grid=(grid,),
        in_specs=[
            pl.BlockSpec((e, _B), lambda i: (0, i)),
            pl.BlockSpec((e, e), lambda i: (0, 0)),
            pl.BlockSpec((e, 1), lambda i: (0, 0)),
        ],
        out_specs=[
            pl.BlockSpec((e, _B), lambda i: (0, i)),
            pl.BlockSpec((1, 1, _B), lambda i: (i, 0, 0)),
        ],
        out_shape=[
            jax.ShapeDtypeStruct((e, batch), jnp.float32),
            jax.ShapeDtypeStruct((grid, 1, _B), jnp.int32),
        ],
    )(xt, attractor_centers, r2d)
    return (out_t.T, idx2.reshape(batch))


# speedup vs baseline: 1.4399x; 1.4399x over previous
import jax
import jax.numpy as jnp
from jax.experimental import pallas as pl

_B = 8192
_E = 64


def _body(xt_ref, c_ref, r_ref, out_ref, idx_ref):
    xt = xt_ref[...]
    c = c_ref[...]
    c_norm = jnp.sum(c * c, axis=1, keepdims=True)
    g = jax.lax.dot_general(
        c, xt, (((1,), (0,)), ((), ())),
        preferred_element_type=jnp.float32,
        precision=jax.lax.Precision.HIGHEST)
    scores = c_norm - 2.0 * g
    m = jnp.min(scores, axis=0, keepdims=True)
    subl = jax.lax.broadcasted_iota(jnp.int32, scores.shape, 0)
    idxrow = jnp.min(jnp.where(scores == m, subl, _E), axis=0,
                     keepdims=True)
    onehot_t = (subl == idxrow).astype(jnp.float32)
    w = 0.1 * jax.nn.sigmoid(r_ref[...])
    rhs = jnp.concatenate([w * c, w], axis=1)
    p = jax.lax.dot_general(
        rhs, onehot_t, (((0,), (0,)), ((), ())),
        preferred_element_type=jnp.float32,
        precision=jax.lax.Precision.DEFAULT)
    out_ref[...] = xt + (p[:_E, :] - p[_E:, :] * xt)
    idx_ref[...] = idxrow[:, None, :]


@jax.jit
def kernel(expert_activations, attractor_centers, attraction_radii):
    batch, e = expert_activations.shape
    xt = expert_activations.T
    grid = batch // _B
    r2d = attraction_radii[:, None]
    out_t, idx2 = pl.pallas_call(
        _body,
        grid=(grid,),
        in_specs=[
            pl.BlockSpec((e, _B), lambda i: (0, i)),
            pl.BlockSpec((e, e), lambda i: (0, 0)),
            pl.BlockSpec((e, 1), lambda i: (0, 0)),
        ],
        out_specs=[
            pl.BlockSpec((e, _B), lambda i: (0, i)),
            pl.BlockSpec((1, 1, _B), lambda i: (i, 0, 0)),
        ],
        out_shape=[
            jax.ShapeDtypeStruct((e, batch), jnp.float32),
            jax.ShapeDtypeStruct((grid, 1, _B), jnp.int32),
        ],
    )(xt, attractor_centers, r2d)
    return (out_t.T, idx2.reshape(batch))
